# Initial kernel scaffold; baseline (speedup 1.0000x reference)
import jax
import jax.numpy as jnp
from jax.experimental import pallas as pl


def kernel(z, edge_index, edge_type, w1_l1, w1_l2, w2_l1, w2_l2):
    raise NotImplementedError("write your pallas kernel here")


# speedup vs baseline: 9.7494x; 9.7494x over previous
import functools

import jax
import jax.numpy as jnp
from jax import lax
from jax.experimental import pallas as pl
from jax.experimental.pallas import tpu as pltpu
from jax.experimental.pallas import tpu_sc as plsc

N_NODES = 10000
N_EDGES = 320000
IN_DIM = 128
L1_DIM = 16
N_TYPES = 1000

N_CORES = 2
N_SUBCORES = 16
N_WORKERS = N_CORES * N_SUBCORES

E_PAD = 327680
EPW = E_PAD // N_WORKERS
CHUNK = 1024
N_CHUNKS = EPW // CHUNK
SUB = CHUNK // 128
GROUPS = CHUNK // 16


def _proj_body(z_ref, w1_ref, w2_ref, p1_ref, p2_ref):
    zz = z_ref[...]
    p1_ref[...] = jnp.maximum(
        jnp.dot(zz, w1_ref[...], preferred_element_type=jnp.float32), 0.0)
    p2_ref[...] = jnp.maximum(
        jnp.dot(zz, w2_ref[...], preferred_element_type=jnp.float32), 0.0)


def _project(z, w1_l1, w2_l1):
    rows = 1000
    grid = (N_NODES // rows,)
    return pl.pallas_call(
        _proj_body,
        grid=grid,
        in_specs=[
            pl.BlockSpec((rows, IN_DIM), lambda i: (i, 0)),
            pl.BlockSpec((IN_DIM, L1_DIM), lambda i: (0, 0)),
            pl.BlockSpec((IN_DIM, L1_DIM), lambda i: (0, 0)),
        ],
        out_specs=[
            pl.BlockSpec((rows, L1_DIM), lambda i: (i, 0)),
            pl.BlockSpec((rows, L1_DIM), lambda i: (i, 0)),
        ],
        out_shape=[
            jax.ShapeDtypeStruct((N_NODES, L1_DIM), jnp.float32),
            jax.ShapeDtypeStruct((N_NODES, L1_DIM), jnp.float32),
        ],
    )(z, w1_l1, w2_l1)


def _edge_body(p1_hbm, p2_hbm, v1_hbm, v2_hbm, src_hbm, dst_hbm, et_hbm,
               out_hbm,
               v1_v, v2_v, sidx_v, didx_v, tidx_v, p1r_v, p2r_v, outb_v, sem):
    wid = lax.axis_index("s") * N_CORES + lax.axis_index("c")
    base = wid * EPW
    row_base = wid * (EPW // 128)

    pltpu.sync_copy(v1_hbm, v1_v)
    pltpu.sync_copy(v2_hbm, v2_v)

    def chunk_body(ci, carry):
        off = base + ci * CHUNK
        roff = row_base + ci * SUB
        pltpu.sync_copy(src_hbm.at[pl.ds(roff, SUB)], sidx_v)
        pltpu.sync_copy(dst_hbm.at[pl.ds(roff, SUB)], didx_v)
        pltpu.sync_copy(et_hbm.at[pl.ds(off, CHUNK)], tidx_v)
        cps = []
        for i in range(SUB):
            cps.append(pltpu.async_copy(
                p1_hbm.at[sidx_v.at[i]], p1r_v.at[pl.ds(i * 128, 128)], sem))
            cps.append(pltpu.async_copy(
                p2_hbm.at[didx_v.at[i]], p2r_v.at[pl.ds(i * 128, 128)], sem))
        for cp in cps:
            cp.wait()

        def group_body(g, gcarry):
            rows = lax.iota(jnp.int32, 16) + g * 16
            tid = tidx_v[pl.ds(g * 16, 16)]
            acc = jnp.zeros((16,), jnp.float32)
            for j in range(L1_DIM):
                jcol = jnp.full((16,), j, jnp.int32)
                a = plsc.load_gather(p1r_v, [rows, jcol])
                b = plsc.load_gather(v1_v, [tid, jcol])
                c = plsc.load_gather(p2r_v, [rows, jcol])
                d = plsc.load_gather(v2_v, [tid, jcol])
                acc = acc + a * b + c * d
            outb_v[pl.ds(g * 16, 16)] = 1.0 / (1.0 + jnp.exp(-acc))
            return gcarry

        lax.fori_loop(0, GROUPS, group_body, 0, unroll=False)
        pltpu.sync_copy(outb_v, out_hbm.at[pl.ds(off, CHUNK)])
        return carry

    lax.fori_loop(0, N_CHUNKS, chunk_body, 0, unroll=False)


_edge_kernel = functools.partial(
    pl.kernel,
    out_type=jax.ShapeDtypeStruct((E_PAD,), jnp.float32),
    mesh=plsc.VectorSubcoreMesh(core_axis_name="c", subcore_axis_name="s"),
    compiler_params=pltpu.CompilerParams(
        needs_layout_passes=False, use_tc_tiling_on_sc=False),
    scratch_types=[
        pltpu.VMEM((N_TYPES, L1_DIM), jnp.float32),
        pltpu.VMEM((N_TYPES, L1_DIM), jnp.float32),
        pltpu.VMEM((SUB, 128), jnp.int32),
        pltpu.VMEM((SUB, 128), jnp.int32),
        pltpu.VMEM((CHUNK,), jnp.int32),
        pltpu.VMEM((CHUNK, L1_DIM), jnp.float32),
        pltpu.VMEM((CHUNK, L1_DIM), jnp.float32),
        pltpu.VMEM((CHUNK,), jnp.float32),
        pltpu.SemaphoreType.DMA,
    ],
)(_edge_body)


def kernel(z, edge_index, edge_type, w1_l1, w1_l2, w2_l1, w2_l2):
    p1, p2 = _project(z, w1_l1, w2_l1)
    src = edge_index[0].astype(jnp.int32)
    dst = edge_index[1].astype(jnp.int32)
    et = edge_type.astype(jnp.int32)
    pad = E_PAD - N_EDGES
    src2d = jnp.pad(src, (0, pad)).reshape(E_PAD // 128, 128)
    dst2d = jnp.pad(dst, (0, pad)).reshape(E_PAD // 128, 128)
    et_p = jnp.pad(et, (0, pad))
    out = _edge_kernel(p1, p2, w1_l2, w2_l2, src2d, dst2d, et_p)
    return out[:N_EDGES]


# speedup vs baseline: 11.0560x; 1.1340x over previous
import functools

import jax
import jax.numpy as jnp
from jax import lax
from jax.experimental import pallas as pl
from jax.experimental.pallas import tpu as pltpu
from jax.experimental.pallas import tpu_sc as plsc

N_NODES = 10000
N_EDGES = 320000
IN_DIM = 128
L1_DIM = 16
N_TYPES = 1000

N_CORES = 2
N_SUBCORES = 16
N_WORKERS = N_CORES * N_SUBCORES

E_PAD = 327680
EPW = E_PAD // N_WORKERS
CHUNK = 1024
N_CHUNKS = EPW // CHUNK
SUB = CHUNK // 128
GROUPS = CHUNK // 16


def _proj_body(z_ref, w1_ref, w2_ref, p1_ref, p2_ref):
    zz = z_ref[...]
    p1_ref[...] = jnp.maximum(
        jnp.dot(zz, w1_ref[...], preferred_element_type=jnp.float32), 0.0)
    p2_ref[...] = jnp.maximum(
        jnp.dot(zz, w2_ref[...], preferred_element_type=jnp.float32), 0.0)


def _project(z, w1_l1, w2_l1):
    rows = 1000
    grid = (N_NODES // rows,)
    return pl.pallas_call(
        _proj_body,
        grid=grid,
        in_specs=[
            pl.BlockSpec((rows, IN_DIM), lambda i: (i, 0)),
            pl.BlockSpec((IN_DIM, L1_DIM), lambda i: (0, 0)),
            pl.BlockSpec((IN_DIM, L1_DIM), lambda i: (0, 0)),
        ],
        out_specs=[
            pl.BlockSpec((rows, L1_DIM), lambda i: (i, 0)),
            pl.BlockSpec((rows, L1_DIM), lambda i: (i, 0)),
        ],
        out_shape=[
            jax.ShapeDtypeStruct((N_NODES, L1_DIM), jnp.float32),
            jax.ShapeDtypeStruct((N_NODES, L1_DIM), jnp.float32),
        ],
    )(z, w1_l1, w2_l1)


def _edge_body(p1_hbm, p2_hbm, v1_hbm, v2_hbm, src_hbm, dst_hbm, et_hbm,
               out_hbm,
               v1_v, v2_v, sidx_v, didx_v, tidx_v, p1r_v, p2r_v, outb_v, sem):
    wid = lax.axis_index("s") * N_CORES + lax.axis_index("c")
    base = wid * EPW
    row_base = wid * (EPW // 128)

    pltpu.sync_copy(v1_hbm, v1_v)
    pltpu.sync_copy(v2_hbm, v2_v)

    def chunk_body(ci, carry):
        off = base + ci * CHUNK
        roff = row_base + ci * SUB
        pltpu.sync_copy(src_hbm.at[pl.ds(roff, SUB)], sidx_v)
        pltpu.sync_copy(dst_hbm.at[pl.ds(roff, SUB)], didx_v)
        pltpu.sync_copy(et_hbm.at[pl.ds(off, CHUNK)], tidx_v)
        cps = []
        for i in range(SUB):
            cps.append(pltpu.async_copy(
                p1_hbm.at[sidx_v.at[i]], p1r_v.at[pl.ds(i * 128, 128)], sem))
            cps.append(pltpu.async_copy(
                p2_hbm.at[didx_v.at[i]], p2r_v.at[pl.ds(i * 128, 128)], sem))
        for cp in cps:
            cp.wait()

        @plsc.parallel_loop(0, GROUPS, unroll=2)
        def group_body(g):
            rows = lax.iota(jnp.int32, 16) + g * 16
            tid = tidx_v[pl.ds(g * 16, 16)]
            acc1 = jnp.zeros((16,), jnp.float32)
            acc2 = jnp.zeros((16,), jnp.float32)
            for j in range(L1_DIM):
                jcol = jnp.full((16,), j, jnp.int32)
                a = plsc.load_gather(p1r_v, [rows, jcol])
                b = plsc.load_gather(v1_v, [tid, jcol])
                c = plsc.load_gather(p2r_v, [rows, jcol])
                d = plsc.load_gather(v2_v, [tid, jcol])
                acc1 = acc1 + a * b
                acc2 = acc2 + c * d
            acc = acc1 + acc2
            outb_v[pl.ds(g * 16, 16)] = 1.0 / (1.0 + jnp.exp(-acc))
        pltpu.sync_copy(outb_v, out_hbm.at[pl.ds(off, CHUNK)])
        return carry

    lax.fori_loop(0, N_CHUNKS, chunk_body, 0, unroll=False)


_edge_kernel = functools.partial(
    pl.kernel,
    out_type=jax.ShapeDtypeStruct((E_PAD,), jnp.float32),
    mesh=plsc.VectorSubcoreMesh(core_axis_name="c", subcore_axis_name="s"),
    compiler_params=pltpu.CompilerParams(
        needs_layout_passes=False, use_tc_tiling_on_sc=False),
    scratch_types=[
        pltpu.VMEM((N_TYPES, L1_DIM), jnp.float32),
        pltpu.VMEM((N_TYPES, L1_DIM), jnp.float32),
        pltpu.VMEM((SUB, 128), jnp.int32),
        pltpu.VMEM((SUB, 128), jnp.int32),
        pltpu.VMEM((CHUNK,), jnp.int32),
        pltpu.VMEM((CHUNK, L1_DIM), jnp.float32),
        pltpu.VMEM((CHUNK, L1_DIM), jnp.float32),
        pltpu.VMEM((CHUNK,), jnp.float32),
        pltpu.SemaphoreType.DMA,
    ],
)(_edge_body)


def kernel(z, edge_index, edge_type, w1_l1, w1_l2, w2_l1, w2_l2):
    p1, p2 = _project(z, w1_l1, w2_l1)
    src = edge_index[0].astype(jnp.int32)
    dst = edge_index[1].astype(jnp.int32)
    et = edge_type.astype(jnp.int32)
    pad = E_PAD - N_EDGES
    src2d = jnp.pad(src, (0, pad)).reshape(E_PAD // 128, 128)
    dst2d = jnp.pad(dst, (0, pad)).reshape(E_PAD // 128, 128)
    et_p = jnp.pad(et, (0, pad))
    out = _edge_kernel(p1, p2, w1_l2, w2_l2, src2d, dst2d, et_p)
    return out[:N_EDGES]


# speedup vs baseline: 11.2431x; 1.0169x over previous
import functools

import jax
import jax.numpy as jnp
from jax import lax
from jax.experimental import pallas as pl
from jax.experimental.pallas import tpu as pltpu
from jax.experimental.pallas import tpu_sc as plsc

N_NODES = 10000
N_EDGES = 320000
IN_DIM = 128
L1_DIM = 16
N_TYPES = 1000

N_CORES = 2
N_SUBCORES = 16
N_WORKERS = N_CORES * N_SUBCORES

E_PAD = 327680
EPW = E_PAD // N_WORKERS
CHUNK = 1024
N_CHUNKS = EPW // CHUNK
SUB = CHUNK // 128
GROUPS = CHUNK // 16


def _scores_body(z_ref, w1_ref, w2_ref, v1_ref, v2_ref, q1_ref, q2_ref):
    zz = z_ref[...]
    p1 = jnp.maximum(
        jnp.dot(zz, w1_ref[...], preferred_element_type=jnp.float32), 0.0)
    p2 = jnp.maximum(
        jnp.dot(zz, w2_ref[...], preferred_element_type=jnp.float32), 0.0)
    q1_ref[...] = jnp.dot(p1, v1_ref[...].T, preferred_element_type=jnp.float32)
    q2_ref[...] = jnp.dot(p2, v2_ref[...].T, preferred_element_type=jnp.float32)


def _scores(z, w1_l1, w2_l1, w1_l2, w2_l2):
    rows = 1000
    grid = (N_NODES // rows,)
    return pl.pallas_call(
        _scores_body,
        grid=grid,
        in_specs=[
            pl.BlockSpec((rows, IN_DIM), lambda i: (i, 0)),
            pl.BlockSpec((IN_DIM, L1_DIM), lambda i: (0, 0)),
            pl.BlockSpec((IN_DIM, L1_DIM), lambda i: (0, 0)),
            pl.BlockSpec((N_TYPES, L1_DIM), lambda i: (0, 0)),
            pl.BlockSpec((N_TYPES, L1_DIM), lambda i: (0, 0)),
        ],
        out_specs=[
            pl.BlockSpec((rows, N_TYPES), lambda i: (i, 0)),
            pl.BlockSpec((rows, N_TYPES), lambda i: (i, 0)),
        ],
        out_shape=[
            jax.ShapeDtypeStruct((N_NODES, N_TYPES), jnp.float32),
            jax.ShapeDtypeStruct((N_NODES, N_TYPES), jnp.float32),
        ],
    )(z, w1_l1, w2_l1, w1_l2, w2_l2)


def _edge_body(q1_hbm, q2_hbm, src_hbm, dst_hbm, et_hbm, out_hbm,
               s_v, d_v, t_v, i1_v, i2_v, q1r_v, q2r_v, outb_v, sem):
    wid = lax.axis_index("s") * N_CORES + lax.axis_index("c")
    base = wid * EPW

    def chunk_body(ci, carry):
        off = base + ci * CHUNK
        pltpu.sync_copy(src_hbm.at[pl.ds(off, CHUNK)], s_v)
        pltpu.sync_copy(dst_hbm.at[pl.ds(off, CHUNK)], d_v)
        pltpu.sync_copy(et_hbm.at[pl.ds(off, CHUNK)], t_v)

        @plsc.parallel_loop(0, GROUPS, unroll=2)
        def idx_body(g):
            sl = pl.ds(g * 16, 16)
            t16 = t_v[sl]
            i1_v[sl] = s_v[sl] * N_TYPES + t16
            i2_v[sl] = d_v[sl] * N_TYPES + t16

        cp1 = pltpu.async_copy(q1_hbm.at[i1_v], q1r_v, sem)
        cp2 = pltpu.async_copy(q2_hbm.at[i2_v], q2r_v, sem)
        cp1.wait()
        cp2.wait()

        @plsc.parallel_loop(0, GROUPS, unroll=2)
        def group_body(g):
            sl = pl.ds(g * 16, 16)
            acc = q1r_v[sl] + q2r_v[sl]
            outb_v[sl] = 1.0 / (1.0 + jnp.exp(-acc))

        pltpu.sync_copy(outb_v, out_hbm.at[pl.ds(off, CHUNK)])
        return carry

    lax.fori_loop(0, N_CHUNKS, chunk_body, 0, unroll=False)


_edge_kernel = functools.partial(
    pl.kernel,
    out_type=jax.ShapeDtypeStruct((E_PAD,), jnp.float32),
    mesh=plsc.VectorSubcoreMesh(core_axis_name="c", subcore_axis_name="s"),
    compiler_params=pltpu.CompilerParams(
        needs_layout_passes=False, use_tc_tiling_on_sc=False),
    scratch_types=[
        pltpu.VMEM((CHUNK,), jnp.int32),
        pltpu.VMEM((CHUNK,), jnp.int32),
        pltpu.VMEM((CHUNK,), jnp.int32),
        pltpu.VMEM((CHUNK,), jnp.int32),
        pltpu.VMEM((CHUNK,), jnp.int32),
        pltpu.VMEM((CHUNK,), jnp.float32),
        pltpu.VMEM((CHUNK,), jnp.float32),
        pltpu.VMEM((CHUNK,), jnp.float32),
        pltpu.SemaphoreType.DMA,
    ],
)(_edge_body)


def kernel(z, edge_index, edge_type, w1_l1, w1_l2, w2_l1, w2_l2):
    q1, q2 = _scores(z, w1_l1, w2_l1, w1_l2, w2_l2)
    q1 = q1.reshape(N_NODES * N_TYPES)
    q2 = q2.reshape(N_NODES * N_TYPES)
    src = edge_index[0].astype(jnp.int32)
    dst = edge_index[1].astype(jnp.int32)
    et = edge_type.astype(jnp.int32)
    pad = E_PAD - N_EDGES
    src_p = jnp.pad(src, (0, pad))
    dst_p = jnp.pad(dst, (0, pad))
    et_p = jnp.pad(et, (0, pad))
    out = _edge_kernel(q1, q2, src_p, dst_p, et_p)
    return out[:N_EDGES]


# speedup vs baseline: 15.8274x; 1.4077x over previous
import functools

import jax
import jax.numpy as jnp
from jax import lax
from jax.experimental import pallas as pl
from jax.experimental.pallas import tpu as pltpu
from jax.experimental.pallas import tpu_sc as plsc

N_NODES = 10000
N_EDGES = 320000
IN_DIM = 128
L1_DIM = 16
N_TYPES = 1000

N_CORES = 2
N_SUBCORES = 16
N_WORKERS = N_CORES * N_SUBCORES

E_PAD = 327680
EPW = E_PAD // N_WORKERS
CHUNK = 1024
N_CHUNKS = EPW // CHUNK
SUB = CHUNK // 128
GROUPS = CHUNK // 16


T_PAD = 1024

_CONTRACT_MINOR = (((1,), (1,)), ((), ()))


def _scores_body(z_ref, w1_ref, w2_ref, v1_ref, v2_ref, q1_ref, q2_ref):
    zz = z_ref[...]
    p1 = jnp.maximum(
        jnp.dot(zz, w1_ref[...], preferred_element_type=jnp.float32), 0.0)
    p2 = jnp.maximum(
        jnp.dot(zz, w2_ref[...], preferred_element_type=jnp.float32), 0.0)
    q1 = lax.dot_general(p1, v1_ref[...], _CONTRACT_MINOR,
                         preferred_element_type=jnp.float32)
    q2 = lax.dot_general(p2, v2_ref[...], _CONTRACT_MINOR,
                         preferred_element_type=jnp.float32)
    for j in range(T_PAD // 128):
        q1_ref[:, j, :] = q1[:, 128 * j:128 * (j + 1)]
        q2_ref[:, j, :] = q2[:, 128 * j:128 * (j + 1)]


def _scores(z, w1_l1, w2_l1, w1_l2, w2_l2):
    rows = 1000
    grid = (N_NODES // rows,)
    v1p = jnp.pad(w1_l2, ((0, T_PAD - N_TYPES), (0, 0)))
    v2p = jnp.pad(w2_l2, ((0, T_PAD - N_TYPES), (0, 0)))
    return pl.pallas_call(
        _scores_body,
        grid=grid,
        in_specs=[
            pl.BlockSpec((rows, IN_DIM), lambda i: (i, 0)),
            pl.BlockSpec((IN_DIM, L1_DIM), lambda i: (0, 0)),
            pl.BlockSpec((IN_DIM, L1_DIM), lambda i: (0, 0)),
            pl.BlockSpec((T_PAD, L1_DIM), lambda i: (0, 0)),
            pl.BlockSpec((T_PAD, L1_DIM), lambda i: (0, 0)),
        ],
        out_specs=[
            pl.BlockSpec((rows, T_PAD // 128, 128), lambda i: (i, 0, 0)),
            pl.BlockSpec((rows, T_PAD // 128, 128), lambda i: (i, 0, 0)),
        ],
        out_shape=[
            jax.ShapeDtypeStruct((N_NODES, T_PAD // 128, 128), jnp.float32),
            jax.ShapeDtypeStruct((N_NODES, T_PAD // 128, 128), jnp.float32),
        ],
    )(z, w1_l1, w2_l1, v1p, v2p)


def _edge_body(q1_hbm, q2_hbm, src_hbm, dst_hbm, et_hbm, out_hbm,
               s_v, d_v, t_v, i1_v, i2_v, q1r_v, q2r_v, outb_v, sem):
    wid = lax.axis_index("s") * N_CORES + lax.axis_index("c")
    base = wid * EPW

    def chunk_body(ci, carry):
        off = base + ci * CHUNK
        pltpu.sync_copy(src_hbm.at[pl.ds(off, CHUNK)], s_v)
        pltpu.sync_copy(dst_hbm.at[pl.ds(off, CHUNK)], d_v)
        pltpu.sync_copy(et_hbm.at[pl.ds(off, CHUNK)], t_v)

        @plsc.parallel_loop(0, GROUPS, unroll=2)
        def idx_body(g):
            sl = pl.ds(g * 16, 16)
            t16 = t_v[sl]
            i1_v[sl] = s_v[sl] * T_PAD + t16
            i2_v[sl] = d_v[sl] * T_PAD + t16

        cp1 = pltpu.async_copy(q1_hbm.at[i1_v], q1r_v, sem)
        cp2 = pltpu.async_copy(q2_hbm.at[i2_v], q2r_v, sem)
        cp1.wait()
        cp2.wait()

        @plsc.parallel_loop(0, GROUPS, unroll=2)
        def group_body(g):
            sl = pl.ds(g * 16, 16)
            acc = q1r_v[sl] + q2r_v[sl]
            outb_v[sl] = 1.0 / (1.0 + jnp.exp(-acc))

        pltpu.sync_copy(outb_v, out_hbm.at[pl.ds(off, CHUNK)])
        return carry

    lax.fori_loop(0, N_CHUNKS, chunk_body, 0, unroll=False)


_edge_kernel = functools.partial(
    pl.kernel,
    out_type=jax.ShapeDtypeStruct((E_PAD,), jnp.float32),
    mesh=plsc.VectorSubcoreMesh(core_axis_name="c", subcore_axis_name="s"),
    compiler_params=pltpu.CompilerParams(
        needs_layout_passes=False, use_tc_tiling_on_sc=False),
    scratch_types=[
        pltpu.VMEM((CHUNK,), jnp.int32),
        pltpu.VMEM((CHUNK,), jnp.int32),
        pltpu.VMEM((CHUNK,), jnp.int32),
        pltpu.VMEM((CHUNK,), jnp.int32),
        pltpu.VMEM((CHUNK,), jnp.int32),
        pltpu.VMEM((CHUNK,), jnp.float32),
        pltpu.VMEM((CHUNK,), jnp.float32),
        pltpu.VMEM((CHUNK,), jnp.float32),
        pltpu.SemaphoreType.DMA,
    ],
)(_edge_body)


def kernel(z, edge_index, edge_type, w1_l1, w1_l2, w2_l1, w2_l2):
    q1, q2 = _scores(z, w1_l1, w2_l1, w1_l2, w2_l2)
    q1 = q1.reshape(N_NODES * T_PAD)
    q2 = q2.reshape(N_NODES * T_PAD)
    src = edge_index[0].astype(jnp.int32)
    dst = edge_index[1].astype(jnp.int32)
    et = edge_type.astype(jnp.int32)
    pad = E_PAD - N_EDGES
    src_p = jnp.pad(src, (0, pad))
    dst_p = jnp.pad(dst, (0, pad))
    et_p = jnp.pad(et, (0, pad))
    out = _edge_kernel(q1, q2, src_p, dst_p, et_p)
    return out[:N_EDGES]


# speedup vs baseline: 17.7127x; 1.1191x over previous
import functools

import jax
import jax.numpy as jnp
from jax import lax
from jax.experimental import pallas as pl
from jax.experimental.pallas import tpu as pltpu
from jax.experimental.pallas import tpu_sc as plsc

N_NODES = 10000
N_EDGES = 320000
IN_DIM = 128
L1_DIM = 16
N_TYPES = 1000

N_CORES = 2
N_SUBCORES = 16
N_WORKERS = N_CORES * N_SUBCORES

E_PAD = 327680
EPW = E_PAD // N_WORKERS
CHUNK = 1024
N_CHUNKS = EPW // CHUNK
SUB = CHUNK // 128
GROUPS = CHUNK // 16


T_PAD = 1024

_CONTRACT_MINOR = (((1,), (1,)), ((), ()))


def _scores_body(z_ref, w1_ref, w2_ref, v1_ref, v2_ref, q1_ref, q2_ref):
    zz = z_ref[...]
    p1 = jnp.maximum(
        jnp.dot(zz, w1_ref[...], preferred_element_type=jnp.float32), 0.0)
    p2 = jnp.maximum(
        jnp.dot(zz, w2_ref[...], preferred_element_type=jnp.float32), 0.0)
    q1 = lax.dot_general(p1, v1_ref[...], _CONTRACT_MINOR,
                         preferred_element_type=jnp.float32)
    q2 = lax.dot_general(p2, v2_ref[...], _CONTRACT_MINOR,
                         preferred_element_type=jnp.float32)
    for j in range(T_PAD // 128):
        q1_ref[:, j, :] = q1[:, 128 * j:128 * (j + 1)]
        q2_ref[:, j, :] = q2[:, 128 * j:128 * (j + 1)]


def _scores(z, w1_l1, w2_l1, w1_l2, w2_l2):
    rows = 1000
    grid = (N_NODES // rows,)
    v1p = jnp.pad(w1_l2, ((0, T_PAD - N_TYPES), (0, 0)))
    v2p = jnp.pad(w2_l2, ((0, T_PAD - N_TYPES), (0, 0)))
    return pl.pallas_call(
        _scores_body,
        grid=grid,
        in_specs=[
            pl.BlockSpec((rows, IN_DIM), lambda i: (i, 0)),
            pl.BlockSpec((IN_DIM, L1_DIM), lambda i: (0, 0)),
            pl.BlockSpec((IN_DIM, L1_DIM), lambda i: (0, 0)),
            pl.BlockSpec((T_PAD, L1_DIM), lambda i: (0, 0)),
            pl.BlockSpec((T_PAD, L1_DIM), lambda i: (0, 0)),
        ],
        out_specs=[
            pl.BlockSpec((rows, T_PAD // 128, 128), lambda i: (i, 0, 0)),
            pl.BlockSpec((rows, T_PAD // 128, 128), lambda i: (i, 0, 0)),
        ],
        out_shape=[
            jax.ShapeDtypeStruct((N_NODES, T_PAD // 128, 128), jnp.float32),
            jax.ShapeDtypeStruct((N_NODES, T_PAD // 128, 128), jnp.float32),
        ],
    )(z, w1_l1, w2_l1, v1p, v2p)


def _edge_body(q1_hbm, q2_hbm, src_hbm, dst_hbm, et_hbm, out_hbm,
               sA, sB, dA, dB, tA, tB, i1A, i1B, i2A, i2B,
               q1A, q1B, q2A, q2B, oA, oB, semA, semB):
    wid = lax.axis_index("s") * N_CORES + lax.axis_index("c")
    base = wid * EPW
    S, D, T = [sA, sB], [dA, dB], [tA, tB]
    I1, I2 = [i1A, i1B], [i2A, i2B]
    Q1R, Q2R, OB = [q1A, q1B], [q2A, q2B], [oA, oB]
    SEM = [semA, semB]

    def fire(c, b):
        off = base + c * CHUNK
        pltpu.sync_copy(src_hbm.at[pl.ds(off, CHUNK)], S[b])
        pltpu.sync_copy(dst_hbm.at[pl.ds(off, CHUNK)], D[b])
        pltpu.sync_copy(et_hbm.at[pl.ds(off, CHUNK)], T[b])

        @plsc.parallel_loop(0, GROUPS, unroll=2)
        def idx_body(g):
            sl = pl.ds(g * 16, 16)
            t16 = T[b][sl]
            I1[b][sl] = S[b][sl] * T_PAD + t16
            I2[b][sl] = D[b][sl] * T_PAD + t16

        return [pltpu.async_copy(q1_hbm.at[I1[b]], Q1R[b], SEM[b]),
                pltpu.async_copy(q2_hbm.at[I2[b]], Q2R[b], SEM[b])]

    pend = [None, None]
    pend[0] = fire(0, 0)
    for c in range(N_CHUNKS):
        b = c & 1
        if c + 1 < N_CHUNKS:
            pend[(c + 1) & 1] = fire(c + 1, (c + 1) & 1)
        for cp in pend[b]:
            cp.wait()

        @plsc.parallel_loop(0, GROUPS, unroll=2)
        def out_body(g):
            sl = pl.ds(g * 16, 16)
            acc = Q1R[b][sl] + Q2R[b][sl]
            OB[b][sl] = 1.0 / (1.0 + jnp.exp(-acc))

        pltpu.sync_copy(OB[b], out_hbm.at[pl.ds(base + c * CHUNK, CHUNK)])


_edge_kernel = functools.partial(
    pl.kernel,
    out_type=jax.ShapeDtypeStruct((E_PAD,), jnp.float32),
    mesh=plsc.VectorSubcoreMesh(core_axis_name="c", subcore_axis_name="s"),
    compiler_params=pltpu.CompilerParams(
        needs_layout_passes=False, use_tc_tiling_on_sc=False),
    scratch_types=(
        [pltpu.VMEM((CHUNK,), jnp.int32) for _ in range(10)]
        + [pltpu.VMEM((CHUNK,), jnp.float32) for _ in range(6)]
        + [pltpu.SemaphoreType.DMA, pltpu.SemaphoreType.DMA]
    ),
)(_edge_body)


def kernel(z, edge_index, edge_type, w1_l1, w1_l2, w2_l1, w2_l2):
    q1, q2 = _scores(z, w1_l1, w2_l1, w1_l2, w2_l2)
    q1 = q1.reshape(N_NODES * T_PAD)
    q2 = q2.reshape(N_NODES * T_PAD)
    src = edge_index[0].astype(jnp.int32)
    dst = edge_index[1].astype(jnp.int32)
    et = edge_type.astype(jnp.int32)
    pad = E_PAD - N_EDGES
    src_p = jnp.pad(src, (0, pad))
    dst_p = jnp.pad(dst, (0, pad))
    et_p = jnp.pad(et, (0, pad))
    out = _edge_kernel(q1, q2, src_p, dst_p, et_p)
    return out[:N_EDGES]


# speedup vs baseline: 25.8202x; 1.4577x over previous
import functools

import jax
import jax.numpy as jnp
from jax import lax
from jax.experimental import pallas as pl
from jax.experimental.pallas import tpu as pltpu
from jax.experimental.pallas import tpu_sc as plsc

N_NODES = 10000
N_EDGES = 320000
IN_DIM = 128
L1_DIM = 16
N_TYPES = 1000
T_PAD = 1024

N_CORES = 2
N_SUBCORES = 16
N_WORKERS = N_CORES * N_SUBCORES

EPW = N_EDGES // N_WORKERS
CHUNK = 2000
N_CHUNKS = EPW // CHUNK
GROUPS_W = EPW // 16
GROUPS_C = CHUNK // 16

_CONTRACT_MINOR = (((1,), (1,)), ((), ()))


def _scores_body(z_ref, w1_ref, w2_ref, v1_ref, v2_ref, q1_ref, q2_ref):
    zz = z_ref[...]
    p1 = jnp.maximum(
        jnp.dot(zz, w1_ref[...], preferred_element_type=jnp.float32), 0.0)
    p2 = jnp.maximum(
        jnp.dot(zz, w2_ref[...], preferred_element_type=jnp.float32), 0.0)
    q1 = lax.dot_general(p1, v1_ref[...], _CONTRACT_MINOR,
                         preferred_element_type=jnp.float32)
    q2 = lax.dot_general(p2, v2_ref[...], _CONTRACT_MINOR,
                         preferred_element_type=jnp.float32)
    for j in range(T_PAD // 128):
        q1_ref[:, j, :] = q1[:, 128 * j:128 * (j + 1)]
        q2_ref[:, j, :] = q2[:, 128 * j:128 * (j + 1)]


def _scores(z, w1_l1, w2_l1, w1_l2, w2_l2):
    rows = 1000
    grid = (N_NODES // rows,)
    v1p = jnp.pad(w1_l2, ((0, T_PAD - N_TYPES), (0, 0)))
    v2p = jnp.pad(w2_l2, ((0, T_PAD - N_TYPES), (0, 0)))
    return pl.pallas_call(
        _scores_body,
        grid=grid,
        in_specs=[
            pl.BlockSpec((rows, IN_DIM), lambda i: (i, 0)),
            pl.BlockSpec((IN_DIM, L1_DIM), lambda i: (0, 0)),
            pl.BlockSpec((IN_DIM, L1_DIM), lambda i: (0, 0)),
            pl.BlockSpec((T_PAD, L1_DIM), lambda i: (0, 0)),
            pl.BlockSpec((T_PAD, L1_DIM), lambda i: (0, 0)),
        ],
        out_specs=[
            pl.BlockSpec((rows, T_PAD // 128, 128), lambda i: (i, 0, 0)),
            pl.BlockSpec((rows, T_PAD // 128, 128), lambda i: (i, 0, 0)),
        ],
        out_shape=[
            jax.ShapeDtypeStruct((N_NODES, T_PAD // 128, 128), jnp.float32),
            jax.ShapeDtypeStruct((N_NODES, T_PAD // 128, 128), jnp.float32),
        ],
    )(z, w1_l1, w2_l1, v1p, v2p)


def _idx_body(src_hbm, dst_hbm, et_hbm, i1_hbm, i2_hbm,
              s_v, d_v, t_v, i1_v, i2_v):
    wid = lax.axis_index("s") * N_CORES + lax.axis_index("c")
    base = wid * EPW
    pltpu.sync_copy(src_hbm.at[pl.ds(base, EPW)], s_v)
    pltpu.sync_copy(dst_hbm.at[pl.ds(base, EPW)], d_v)
    pltpu.sync_copy(et_hbm.at[pl.ds(base, EPW)], t_v)

    @plsc.parallel_loop(0, GROUPS_W, unroll=4)
    def idx_body(g):
        sl = pl.ds(g * 16, 16)
        t16 = t_v[sl]
        i1_v[sl] = s_v[sl] * T_PAD + t16
        i2_v[sl] = d_v[sl] * T_PAD + t16

    pltpu.sync_copy(i1_v, i1_hbm.at[pl.ds(base, EPW)])
    pltpu.sync_copy(i2_v, i2_hbm.at[pl.ds(base, EPW)])


_idx_kernel = functools.partial(
    pl.kernel,
    out_type=[jax.ShapeDtypeStruct((N_EDGES,), jnp.int32),
              jax.ShapeDtypeStruct((N_EDGES,), jnp.int32)],
    mesh=plsc.VectorSubcoreMesh(core_axis_name="c", subcore_axis_name="s"),
    compiler_params=pltpu.CompilerParams(
        needs_layout_passes=False, use_tc_tiling_on_sc=False),
    scratch_types=[pltpu.VMEM((EPW,), jnp.int32) for _ in range(5)],
)(_idx_body)


def _edge_body(q1_hbm, q2_hbm, i1_hbm, i2_hbm, out_hbm,
               i1_v, i2_v, q1r_v, q2r_v, outb_v, sem):
    wid = lax.axis_index("s") * N_CORES + lax.axis_index("c")
    base = wid * EPW
    pltpu.sync_copy(i1_hbm.at[pl.ds(base, EPW)], i1_v)
    pltpu.sync_copy(i2_hbm.at[pl.ds(base, EPW)], i2_v)

    pend = []
    for c in range(N_CHUNKS):
        sl = pl.ds(c * CHUNK, CHUNK)
        pend.append([
            pltpu.async_copy(q1_hbm.at[i1_v.at[sl]], q1r_v.at[sl], sem),
            pltpu.async_copy(q2_hbm.at[i2_v.at[sl]], q2r_v.at[sl], sem),
        ])
    for c in range(N_CHUNKS):
        for cp in pend[c]:
            cp.wait()

        @plsc.parallel_loop(0, GROUPS_C, unroll=2)
        def out_body(g):
            sl = pl.ds(c * CHUNK + g * 16, 16)
            acc = q1r_v[sl] + q2r_v[sl]
            outb_v[sl] = 1.0 / (1.0 + jnp.exp(-acc))

        pltpu.sync_copy(outb_v.at[pl.ds(c * CHUNK, CHUNK)],
                        out_hbm.at[pl.ds(base + c * CHUNK, CHUNK)])


_edge_kernel = functools.partial(
    pl.kernel,
    out_type=jax.ShapeDtypeStruct((N_EDGES,), jnp.float32),
    mesh=plsc.VectorSubcoreMesh(core_axis_name="c", subcore_axis_name="s"),
    compiler_params=pltpu.CompilerParams(
        needs_layout_passes=False, use_tc_tiling_on_sc=False),
    scratch_types=[
        pltpu.VMEM((EPW,), jnp.int32),
        pltpu.VMEM((EPW,), jnp.int32),
        pltpu.VMEM((EPW,), jnp.float32),
        pltpu.VMEM((EPW,), jnp.float32),
        pltpu.VMEM((EPW,), jnp.float32),
        pltpu.SemaphoreType.DMA,
    ],
)(_edge_body)


def kernel(z, edge_index, edge_type, w1_l1, w1_l2, w2_l1, w2_l2):
    src = edge_index[0].astype(jnp.int32)
    dst = edge_index[1].astype(jnp.int32)
    et = edge_type.astype(jnp.int32)
    i1, i2 = _idx_kernel(src, dst, et)
    q1, q2 = _scores(z, w1_l1, w2_l1, w1_l2, w2_l2)
    q1 = q1.reshape(N_NODES * T_PAD)
    q2 = q2.reshape(N_NODES * T_PAD)
    return _edge_kernel(q1, q2, i1, i2)


# speedup vs baseline: 26.9316x; 1.0430x over previous
import functools

import jax
import jax.numpy as jnp
from jax import lax
from jax.experimental import pallas as pl
from jax.experimental.pallas import tpu as pltpu
from jax.experimental.pallas import tpu_sc as plsc

N_NODES = 10000
N_EDGES = 320000
IN_DIM = 128
L1_DIM = 16
N_TYPES = 1000
T_PAD = 1024

N_CORES = 2
N_SUBCORES = 16
N_WORKERS = N_CORES * N_SUBCORES

EPW = N_EDGES // N_WORKERS
CHUNK = 2000
N_CHUNKS = EPW // CHUNK
GROUPS_W = EPW // 16
GROUPS_C = CHUNK // 16

_CONTRACT_MINOR = (((1,), (1,)), ((), ()))

_SC_MESH = plsc.VectorSubcoreMesh(core_axis_name="c", subcore_axis_name="s")
_SC_PARAMS = pltpu.CompilerParams(
    needs_layout_passes=False, use_tc_tiling_on_sc=False)


def _score_body(z_ref, w_ref, v_ref, q_ref):
    p = jnp.maximum(
        jnp.dot(z_ref[...], w_ref[...], preferred_element_type=jnp.float32),
        0.0)
    q = lax.dot_general(p, v_ref[...], _CONTRACT_MINOR,
                        preferred_element_type=jnp.float32)
    for j in range(T_PAD // 128):
        q_ref[:, j, :] = q[:, 128 * j:128 * (j + 1)]


def _scores(z, w_l1, w_l2):
    rows = 1000
    grid = (N_NODES // rows,)
    vp = jnp.pad(w_l2, ((0, T_PAD - N_TYPES), (0, 0)))
    q = pl.pallas_call(
        _score_body,
        grid=grid,
        in_specs=[
            pl.BlockSpec((rows, IN_DIM), lambda i: (i, 0)),
            pl.BlockSpec((IN_DIM, L1_DIM), lambda i: (0, 0)),
            pl.BlockSpec((T_PAD, L1_DIM), lambda i: (0, 0)),
        ],
        out_specs=pl.BlockSpec((rows, T_PAD // 128, 128), lambda i: (i, 0, 0)),
        out_shape=jax.ShapeDtypeStruct((N_NODES, T_PAD // 128, 128),
                                       jnp.float32),
    )(z, w_l1, vp)
    return q.reshape(N_NODES * T_PAD)


def _idx_body(src_hbm, dst_hbm, et_hbm, i1_hbm, i2_hbm,
              s_v, d_v, t_v, i1_v, i2_v):
    wid = lax.axis_index("s") * N_CORES + lax.axis_index("c")
    base = wid * EPW
    pltpu.sync_copy(src_hbm.at[pl.ds(base, EPW)], s_v)
    pltpu.sync_copy(dst_hbm.at[pl.ds(base, EPW)], d_v)
    pltpu.sync_copy(et_hbm.at[pl.ds(base, EPW)], t_v)

    @plsc.parallel_loop(0, GROUPS_W, unroll=4)
    def idx_body(g):
        sl = pl.ds(g * 16, 16)
        t16 = t_v[sl]
        i1_v[sl] = s_v[sl] * T_PAD + t16
        i2_v[sl] = d_v[sl] * T_PAD + t16

    pltpu.sync_copy(i1_v, i1_hbm.at[pl.ds(base, EPW)])
    pltpu.sync_copy(i2_v, i2_hbm.at[pl.ds(base, EPW)])


_idx_kernel = functools.partial(
    pl.kernel,
    out_type=[jax.ShapeDtypeStruct((N_EDGES,), jnp.int32),
              jax.ShapeDtypeStruct((N_EDGES,), jnp.int32)],
    mesh=_SC_MESH,
    compiler_params=_SC_PARAMS,
    scratch_types=[pltpu.VMEM((EPW,), jnp.int32) for _ in range(5)],
)(_idx_body)


def _gather1_body(q1_hbm, i1_hbm, part_hbm, i1_v, q1r_v, sem):
    wid = lax.axis_index("s") * N_CORES + lax.axis_index("c")
    base = wid * EPW
    pltpu.sync_copy(i1_hbm.at[pl.ds(base, EPW)], i1_v)
    pend = []
    for c in range(N_CHUNKS):
        sl = pl.ds(c * CHUNK, CHUNK)
        pend.append(pltpu.async_copy(q1_hbm.at[i1_v.at[sl]], q1r_v.at[sl],
                                     sem))
    for c in range(N_CHUNKS):
        pend[c].wait()
        pltpu.sync_copy(q1r_v.at[pl.ds(c * CHUNK, CHUNK)],
                        part_hbm.at[pl.ds(base + c * CHUNK, CHUNK)])


_gather1_kernel = functools.partial(
    pl.kernel,
    out_type=jax.ShapeDtypeStruct((N_EDGES,), jnp.float32),
    mesh=_SC_MESH,
    compiler_params=_SC_PARAMS,
    scratch_types=[
        pltpu.VMEM((EPW,), jnp.int32),
        pltpu.VMEM((EPW,), jnp.float32),
        pltpu.SemaphoreType.DMA,
    ],
)(_gather1_body)


def _gather2_body(q2_hbm, i2_hbm, part_hbm, out_hbm,
                  i2_v, q2r_v, partb_v, outb_v, sem):
    wid = lax.axis_index("s") * N_CORES + lax.axis_index("c")
    base = wid * EPW
    pltpu.sync_copy(i2_hbm.at[pl.ds(base, EPW)], i2_v)
    pend = []
    for c in range(N_CHUNKS):
        sl = pl.ds(c * CHUNK, CHUNK)
        pend.append(pltpu.async_copy(q2_hbm.at[i2_v.at[sl]], q2r_v.at[sl],
                                     sem))
    pltpu.sync_copy(part_hbm.at[pl.ds(base, EPW)], partb_v)
    for c in range(N_CHUNKS):
        pend[c].wait()

        @plsc.parallel_loop(0, GROUPS_C, unroll=2)
        def out_body(g):
            sl = pl.ds(c * CHUNK + g * 16, 16)
            acc = partb_v[sl] + q2r_v[sl]
            outb_v[sl] = 1.0 / (1.0 + jnp.exp(-acc))

        pltpu.sync_copy(outb_v.at[pl.ds(c * CHUNK, CHUNK)],
                        out_hbm.at[pl.ds(base + c * CHUNK, CHUNK)])


_gather2_kernel = functools.partial(
    pl.kernel,
    out_type=jax.ShapeDtypeStruct((N_EDGES,), jnp.float32),
    mesh=_SC_MESH,
    compiler_params=_SC_PARAMS,
    scratch_types=[
        pltpu.VMEM((EPW,), jnp.int32),
        pltpu.VMEM((EPW,), jnp.float32),
        pltpu.VMEM((EPW,), jnp.float32),
        pltpu.VMEM((EPW,), jnp.float32),
        pltpu.SemaphoreType.DMA,
    ],
)(_gather2_body)


def kernel(z, edge_index, edge_type, w1_l1, w1_l2, w2_l1, w2_l2):
    src = edge_index[0].astype(jnp.int32)
    dst = edge_index[1].astype(jnp.int32)
    et = edge_type.astype(jnp.int32)
    i1, i2 = _idx_kernel(src, dst, et)
    q1 = _scores(z, w1_l1, w1_l2)
    part = _gather1_kernel(q1, i1)
    q2 = _scores(z, w2_l1, w2_l2)
    return _gather2_kernel(q2, i2, part)
